# baseline (speedup 1.0000x reference)
import functools

import jax
import jax.numpy as jnp
from jax import lax
from jax.experimental import pallas as pl
from jax.experimental.pallas import tpu as pltpu
from jax.experimental.pallas import tpu_sc as plsc

N = 10000
E = 320000
F_IN = 128
H = 8
C = 16
HC = H * C
B = 64
G = 16
NCLS = 10

NCORE = 2
NSUB = 16
LN = 16
NW = NCORE * NSUB

CHUNK = 64
GRP = 6
NGRP = 2 * (-(-E // (NW * CHUNK * GRP * 2)))
NCHUNK = NGRP * GRP
EPW = CHUNK * NCHUNK
EPAD = EPW * NW
NP = 10240
RPT = NP // NSUB
NEB = CHUNK // LN


_BN = 400


def _pre_body(x0, x1, x2, w0, w1, w2, m0, m1, m2,
              h0, h1, h2, ab0, ab1, ab2):
    for x, w, m, h, ab in ((x0, w0, m0, h0, ab0),
                           (x1, w1, m1, h1, ab1),
                           (x2, w2, m2, h2, ab2)):
        hv = jnp.dot(x[...], w[...], preferred_element_type=jnp.float32)
        h[...] = hv
        ab[...] = jnp.dot(hv, m[...], preferred_element_type=jnp.float32)


def _tc_pre(xs, Ws, ABs):
    grid = (N // _BN,)
    xspec = pl.BlockSpec((_BN, F_IN), lambda i: (i, 0))
    wspec = pl.BlockSpec((F_IN, HC), lambda i: (0, 0))
    mspec = pl.BlockSpec((HC, 16), lambda i: (0, 0))
    hspec = pl.BlockSpec((_BN, HC), lambda i: (i, 0))
    abspec = pl.BlockSpec((_BN, 16), lambda i: (i, 0))
    out_shape = ([jax.ShapeDtypeStruct((N, HC), jnp.float32) for _ in range(3)]
                 + [jax.ShapeDtypeStruct((N, 16), jnp.float32) for _ in range(3)])
    res = pl.pallas_call(
        _pre_body,
        grid=grid,
        in_specs=[xspec] * 3 + [wspec] * 3 + [mspec] * 3,
        out_specs=[hspec] * 3 + [abspec] * 3,
        out_shape=out_shape,
    )(*xs, *Ws, *ABs)
    return res[:3], res[3:]


def _sc_body(h0, h1, h2, ab0, ab1, ab2, e0, e1, e2,
             num_out, den_out,
             acc_num, acc_den, idxg, absrc, abdst, hrows,
             exden, sg0, sg1, sg2, ss0, ss1, si0, si1):
    cid = lax.axis_index("c")
    sid = lax.axis_index("s")
    w = cid * NSUB + sid
    iota = lax.broadcasted_iota(jnp.int32, (LN,), 0)
    zero16 = jnp.zeros((LN,), jnp.float32)
    rows0 = sid * RPT
    sg = (sg0, sg1, sg2)
    ss = (ss0, ss1)
    si = (si0, si1)

    def issue_gathers(abv, hv, hslot, aslot, src_row, dst_row):
        pltpu.async_copy(abv.at[src_row], absrc.at[aslot], sg[hslot])
        pltpu.async_copy(abv.at[dst_row], abdst.at[aslot], sg[hslot])
        pltpu.async_copy(hv.at[src_row], hrows.at[hslot], sg[hslot])

    def wait_gathers(abv, hv, hslot, aslot, src_row, dst_row):
        pltpu.make_async_copy(abv.at[src_row], absrc.at[aslot],
                              sg[hslot]).wait()
        pltpu.make_async_copy(abv.at[dst_row], abdst.at[aslot],
                              sg[hslot]).wait()
        pltpu.make_async_copy(hv.at[src_row], hrows.at[hslot],
                              sg[hslot]).wait()

    def issue_scatters(hslot, eslot, dst_row):
        pltpu.async_copy(hrows.at[hslot], acc_num.at[dst_row], ss[eslot],
                         add=True)
        pltpu.async_copy(exden.at[eslot], acc_den.at[dst_row], ss[eslot],
                         add=True)

    def wait_scatters(hslot, eslot, dst_row):
        pltpu.make_async_copy(hrows.at[hslot], acc_num.at[dst_row],
                              ss[eslot]).wait()
        pltpu.make_async_copy(exden.at[eslot], acc_den.at[dst_row],
                              ss[eslot]).wait()

    eis = [iota + (eb * LN) for eb in range(NEB)]

    def compute_chunk(hslot, aslot):
        def _ph(h, _):
            hcol = jnp.full((LN,), h, jnp.int32)
            hcol8 = hcol + 8
            exh = []
            for eb in range(NEB):
                va = plsc.load_gather(absrc.at[aslot], [eis[eb], hcol])
                vb = plsc.load_gather(abdst.at[aslot], [eis[eb], hcol8])
                al = va + vb
                al = jnp.where(al > 0, al, al * jnp.float32(0.2))
                ex = jnp.exp(al)
                plsc.store_scatter(exden.at[aslot], [eis[eb], hcol], ex)
                exh.append(ex)
            hbase = jnp.full((LN,), h * 16, jnp.int32)

            @plsc.parallel_loop(0, 16, unroll=8)
            def _col(c):
                colv = hbase + c
                for eb in range(NEB):
                    hvv = plsc.load_gather(hrows.at[hslot], [eis[eb], colv])
                    plsc.store_scatter(hrows.at[hslot], [eis[eb], colv],
                                       hvv * exh[eb])

            return 0

        lax.fori_loop(0, H, _ph, 0)

    for v, (hv, abv, ev) in enumerate(
            ((h0, ab0, e0), (h1, ab1, e1), (h2, ab2, e2))):
        def _zb(r, _):
            for cc in range(8):
                hrows[0, r, pl.ds(cc * 16, 16)] = zero16
            exden[0, r, :] = zero16
            return 0

        lax.fori_loop(0, CHUNK, _zb, 0)
        for k in range(RPT // CHUNK):
            pltpu.sync_copy(hrows.at[0],
                            acc_num.at[pl.ds(rows0 + k * CHUNK, CHUNK), :])
            pltpu.sync_copy(exden.at[0],
                            acc_den.at[pl.ds(rows0 + k * CHUNK, CHUNK), :])
        plsc.subcore_barrier()

        pltpu.async_copy(ev.at[w, 0], idxg.at[0], si[0])
        pltpu.make_async_copy(ev.at[w, 0], idxg.at[0], si[0]).wait()
        issue_gathers(abv, hv, 0, 0, idxg.at[0, 0, 0], idxg.at[0, 0, 1])

        def _grp2(i, _):
          for gp in range(2):
            g = 2 * i + gp
            s_cur = gp
            s_nxt = 1 - gp
            for b in range(GRP):
                hsl = b % 3
                asl = b % 2
                if b <= 1:
                    @pl.when(g > 0)
                    def _():
                        wait_scatters((b + 4) % 3, b % 2,
                                      idxg.at[s_nxt, b + 4, 1])
                else:
                    wait_scatters((b - 2) % 3, b % 2,
                                  idxg.at[s_cur, b - 2, 1])
                if b == 1:
                    @pl.when(g < NGRP - 1)
                    def _():
                        pltpu.async_copy(ev.at[w, g + 1], idxg.at[s_nxt],
                                         si[s_nxt])
                if b == GRP - 1:
                    @pl.when(g < NGRP - 1)
                    def _():
                        pltpu.make_async_copy(ev.at[w, g + 1], idxg.at[s_nxt],
                                              si[s_nxt]).wait()
                        issue_gathers(abv, hv, 0, 0, idxg.at[s_nxt, 0, 0],
                                      idxg.at[s_nxt, 0, 1])
                else:
                    issue_gathers(abv, hv, (b + 1) % 3, (b + 1) % 2,
                                  idxg.at[s_cur, b + 1, 0],
                                  idxg.at[s_cur, b + 1, 1])
                wait_gathers(abv, hv, hsl, asl, idxg.at[s_cur, b, 0],
                             idxg.at[s_cur, b, 1])
                compute_chunk(hsl, asl)
                issue_scatters(hsl, asl, idxg.at[s_cur, b, 1])
          return 0

        lax.fori_loop(0, NGRP // 2, _grp2, 0)
        wait_scatters((GRP - 2) % 3, (GRP - 2) % 2,
                      idxg.at[(NGRP - 1) % 2, GRP - 2, 1])
        wait_scatters((GRP - 1) % 3, (GRP - 1) % 2,
                      idxg.at[(NGRP - 1) % 2, GRP - 1, 1])
        plsc.subcore_barrier()

        for k in range(RPT // CHUNK):
            pltpu.sync_copy(
                acc_num.at[pl.ds(rows0 + k * CHUNK, CHUNK), :],
                num_out.at[v, cid, pl.ds(rows0 + k * CHUNK, CHUNK), :])
        pltpu.sync_copy(acc_den.at[pl.ds(rows0, RPT), :],
                        den_out.at[v, cid, pl.ds(rows0, RPT), :])
        plsc.subcore_barrier()


def _sc_edge(hs, abs_pad, edges):
    mesh = plsc.VectorSubcoreMesh(core_axis_name="c", subcore_axis_name="s")
    f32 = jnp.float32
    kern = pl.kernel(
        _sc_body,
        out_type=(jax.ShapeDtypeStruct((3, NCORE, NP, HC), f32),
                  jax.ShapeDtypeStruct((3, NCORE, NP, 16), f32)),
        mesh=mesh,
        compiler_params=pltpu.CompilerParams(
            needs_layout_passes=False, use_tc_tiling_on_sc=False),
        scratch_types=[
            pltpu.VMEM_SHARED((NP, HC), f32),
            pltpu.VMEM_SHARED((NP, 16), f32),
            pltpu.VMEM((2, GRP, 2, CHUNK), jnp.int32),
            pltpu.VMEM((2, CHUNK, 16), f32),
            pltpu.VMEM((2, CHUNK, 16), f32),
            pltpu.VMEM((3, CHUNK, HC), f32),
            pltpu.VMEM((2, CHUNK, 16), f32),
            pltpu.SemaphoreType.DMA,
            pltpu.SemaphoreType.DMA,
            pltpu.SemaphoreType.DMA,
            pltpu.SemaphoreType.DMA,
            pltpu.SemaphoreType.DMA,
            pltpu.SemaphoreType.DMA,
            pltpu.SemaphoreType.DMA,
        ],
    )
    return kern(*hs, *abs_pad, *edges)


_PB = 1000


def _post_body(num, den, h0, h1, h2, ab0, ab1, ab2, batch3, rep,
               gfeat, b0, b1, b2, lin1w, lin1b, lin2w, lin2b,
               out, accp, acccnt, xcat):
    i = pl.program_id(0)
    nsteps = pl.num_programs(0)

    @pl.when(i == 0)
    def _init():
        accp[...] = jnp.zeros((B, 3 * HC), jnp.float32)
        acccnt[...] = jnp.zeros((B, HC), jnp.float32)
        xcat[...] = jnp.zeros((B, 512), jnp.float32)

    bb = batch3[0]
    iota_b = lax.broadcasted_iota(jnp.int32, (B, _PB), 0)
    oh = (iota_b == bb).astype(jnp.float32)
    acccnt[...] += jnp.sum(oh, axis=1, keepdims=True)

    repm = rep[...]
    for v, (hr, abr) in enumerate(((h0, ab0), (h1, ab1), (h2, ab2))):
        nb = num[v, 0] + num[v, 1]
        db8 = den[v, 0, :, :H] + den[v, 1, :, :H]
        abb = abr[...]
        a8 = abb[:, :H] + abb[:, H:2 * H]
        a128 = jnp.dot(a8, repm, preferred_element_type=jnp.float32)
        el128 = jnp.exp(jnp.where(a128 > 0, a128, a128 * jnp.float32(0.2)))
        d128 = jnp.dot(db8, repm, preferred_element_type=jnp.float32) + el128
        nfull = nb + hr[...] * el128
        outv = nfull / (d128 + jnp.float32(1e-16))
        accp[:, v * HC:(v + 1) * HC] += jnp.dot(
            oh, outv, preferred_element_type=jnp.float32)

    @pl.when(i == nsteps - 1)
    def _fin():
        cnt = acccnt[...]
        denom = jnp.maximum(cnt, 1.0)
        nonempty = (cnt > 0).astype(jnp.float32)
        for v, bv in enumerate((b0, b1, b2)):
            xcat[:, v * HC:(v + 1) * HC] = (
                accp[:, v * HC:(v + 1) * HC] / denom + bv[...] * nonempty)
        xcat[:, 3 * HC:3 * HC + G] = gfeat[...]
        y1 = jnp.dot(xcat[...], lin1w[...], preferred_element_type=jnp.float32)
        y1 = jnp.maximum(y1 + lin1b[...], 0.0)
        y = jnp.dot(y1, lin2w[...], preferred_element_type=jnp.float32)
        out[...] = y + lin2b[...]


def _tc_post(num, den, hs, abs_, batch3, rep, gfeat, biases,
             lin1w_pad, lin1b, lin2w, lin2b):
    grid = (N // _PB,)
    f32 = jnp.float32
    const = lambda shape: pl.BlockSpec(shape, lambda i: tuple(0 for _ in shape))
    in_specs = [
        pl.BlockSpec((3, NCORE, _PB, HC), lambda i: (0, 0, i, 0)),
        pl.BlockSpec((3, NCORE, _PB, 16), lambda i: (0, 0, i, 0)),
    ]
    in_specs += [pl.BlockSpec((_PB, HC), lambda i: (i, 0))] * 3
    in_specs += [pl.BlockSpec((_PB, 16), lambda i: (i, 0))] * 3
    in_specs += [pl.BlockSpec((1, 1, _PB), lambda i: (i, 0, 0))]
    in_specs += [const((H, HC)), const((B, G)),
                 const((1, HC)), const((1, HC)), const((1, HC)),
                 const((512, 16)), const((1, 16)), const((16, NCLS)),
                 const((1, NCLS))]
    out = pl.pallas_call(
        _post_body,
        grid=grid,
        in_specs=in_specs,
        out_specs=pl.BlockSpec((B, NCLS), lambda i: (0, 0)),
        out_shape=jax.ShapeDtypeStruct((B, NCLS), f32),
        scratch_shapes=[
            pltpu.VMEM((B, 3 * HC), f32),
            pltpu.VMEM((B, HC), f32),
            pltpu.VMEM((B, 512), f32),
        ],
    )(num, den, *hs, *abs_, batch3, rep, gfeat, *biases,
      lin1w_pad, lin1b, lin2w, lin2b)
    return out


def kernel(x_anterior, x_posterior, x_lateral, global_features,
           edge_index_anterior, edge_index_posterior, edge_index_lateral,
           batch,
           W_ant, att_src_ant, att_dst_ant, bias_ant,
           W_post, att_src_post, att_dst_post, bias_post,
           W_lat, att_src_lat, att_dst_lat, bias_lat,
           lin1_w, lin1_b, lin2_w, lin2_b):
    f32 = jnp.float32

    def make_ab_mat(att_src, att_dst):
        eye = jnp.eye(H, dtype=f32)
        sel = jnp.repeat(eye, C, axis=0)
        m_src = sel * att_src.reshape(HC, 1)
        m_dst = sel * att_dst.reshape(HC, 1)
        return jnp.concatenate([m_src, m_dst], axis=1)

    ABs = [make_ab_mat(att_src_ant, att_dst_ant),
           make_ab_mat(att_src_post, att_dst_post),
           make_ab_mat(att_src_lat, att_dst_lat)]

    xs = [x_anterior, x_posterior, x_lateral]
    Ws = [W_ant, W_post, W_lat]

    hs, abs_ = _tc_pre(xs, Ws, ABs)

    abs_pad = [jnp.pad(a, ((0, NP - N), (0, 0))) for a in abs_]

    def prep_edges(ei):
        src = jnp.concatenate(
            [ei[0].astype(jnp.int32), jnp.zeros((EPAD - E,), jnp.int32)])
        dst = jnp.concatenate(
            [ei[1].astype(jnp.int32),
             jnp.full((EPAD - E,), N, jnp.int32)])
        src4 = src.reshape(NW, NGRP, GRP, 1, CHUNK)
        dst4 = dst.reshape(NW, NGRP, GRP, 1, CHUNK)
        return jnp.concatenate([src4, dst4], axis=3)

    edges = [prep_edges(edge_index_anterior),
             prep_edges(edge_index_posterior),
             prep_edges(edge_index_lateral)]

    num, den = _sc_edge(hs, abs_pad, edges)

    repm = jnp.repeat(jnp.eye(H, dtype=f32), C, axis=0).T

    batch3 = batch.astype(jnp.int32).reshape(N // _PB, 1, _PB)
    lin1w_pad = jnp.pad(lin1_w, ((0, 512 - lin1_w.shape[0]), (0, 0)))
    biases = [bias_ant.reshape(1, HC), bias_post.reshape(1, HC),
              bias_lat.reshape(1, HC)]

    out = _tc_post(num, den, hs, abs_, batch3, repm,
                   global_features, biases, lin1w_pad,
                   lin1_b.reshape(1, 16), lin2_w, lin2_b.reshape(1, NCLS))
    return out


# speedup vs baseline: 1.1934x; 1.1934x over previous
import functools

import jax
import jax.numpy as jnp
from jax import lax
from jax.experimental import pallas as pl
from jax.experimental.pallas import tpu as pltpu
from jax.experimental.pallas import tpu_sc as plsc

N = 10000
E = 320000
F_IN = 128
H = 8
C = 16
HC = H * C
B = 64
G = 16
NCLS = 10

NCORE = 2
NSUB = 16
LN = 16
NW = NCORE * NSUB

CHUNK = 80
GRP = 4
NGRP = 2 * (-(-E // (NW * CHUNK * GRP * 2)))
NCHUNK = NGRP * GRP
EPW = CHUNK * NCHUNK
EPAD = EPW * NW
NP = 10240
RPT = NP // NSUB
NEB = CHUNK // LN


_BN = 400


def _pre_body(x0, x1, x2, w0, w1, w2, m0, m1, m2,
              h0, h1, h2, ab0, ab1, ab2):
    for x, w, m, h, ab in ((x0, w0, m0, h0, ab0),
                           (x1, w1, m1, h1, ab1),
                           (x2, w2, m2, h2, ab2)):
        hv = jnp.dot(x[...], w[...], preferred_element_type=jnp.float32)
        h[...] = hv
        ab[...] = jnp.dot(hv, m[...], preferred_element_type=jnp.float32)


def _tc_pre(xs, Ws, ABs):
    grid = (N // _BN,)
    xspec = pl.BlockSpec((_BN, F_IN), lambda i: (i, 0))
    wspec = pl.BlockSpec((F_IN, HC), lambda i: (0, 0))
    mspec = pl.BlockSpec((HC, 16), lambda i: (0, 0))
    hspec = pl.BlockSpec((_BN, HC), lambda i: (i, 0))
    abspec = pl.BlockSpec((_BN, 16), lambda i: (i, 0))
    out_shape = ([jax.ShapeDtypeStruct((N, HC), jnp.float32) for _ in range(3)]
                 + [jax.ShapeDtypeStruct((N, 16), jnp.float32) for _ in range(3)])
    res = pl.pallas_call(
        _pre_body,
        grid=grid,
        in_specs=[xspec] * 3 + [wspec] * 3 + [mspec] * 3,
        out_specs=[hspec] * 3 + [abspec] * 3,
        out_shape=out_shape,
    )(*xs, *Ws, *ABs)
    return res[:3], res[3:]


def _sc_body(h0, h1, h2, ab0, ab1, ab2, e0, e1, e2,
             num_out, den_out,
             acc_num, acc_den, idxg, absrc, abdst, hrows,
             exden, sg0, sg1, ss0, ss1, si0, si1):
    cid = lax.axis_index("c")
    sid = lax.axis_index("s")
    w = cid * NSUB + sid
    iota = lax.broadcasted_iota(jnp.int32, (LN,), 0)
    zero16 = jnp.zeros((LN,), jnp.float32)
    rows0 = sid * RPT
    sg = (sg0, sg1)
    ss = (ss0, ss1)
    si = (si0, si1)

    def issue_gathers(abv, hv, slot, src_row, dst_row):
        pltpu.async_copy(abv.at[src_row], absrc.at[slot], sg[slot])
        pltpu.async_copy(abv.at[dst_row], abdst.at[slot], sg[slot])
        pltpu.async_copy(hv.at[src_row], hrows.at[slot], sg[slot])

    def wait_gathers(abv, hv, slot, src_row, dst_row):
        pltpu.make_async_copy(abv.at[src_row], absrc.at[slot],
                              sg[slot]).wait()
        pltpu.make_async_copy(abv.at[dst_row], abdst.at[slot],
                              sg[slot]).wait()
        pltpu.make_async_copy(hv.at[src_row], hrows.at[slot], sg[slot]).wait()

    def issue_scatters(slot, dst_row):
        pltpu.async_copy(hrows.at[slot], acc_num.at[dst_row], ss[slot],
                         add=True)
        pltpu.async_copy(exden, acc_den.at[dst_row], ss[slot], add=True)

    def wait_scatters(slot, dst_row):
        pltpu.make_async_copy(hrows.at[slot], acc_num.at[dst_row],
                              ss[slot]).wait()
        pltpu.make_async_copy(exden, acc_den.at[dst_row], ss[slot]).wait()

    eis = [iota + (eb * LN) for eb in range(NEB)]

    def compute_chunk(slot):
        def _ph(h, _):
            hcol = jnp.full((LN,), h, jnp.int32)
            hcol8 = hcol + 8
            exh = []
            for eb in range(NEB):
                va = plsc.load_gather(absrc.at[slot], [eis[eb], hcol])
                vb = plsc.load_gather(abdst.at[slot], [eis[eb], hcol8])
                al = va + vb
                al = jnp.where(al > 0, al, al * jnp.float32(0.2))
                ex = jnp.exp(al)
                plsc.store_scatter(exden, [eis[eb], hcol], ex)
                exh.append(ex)
            hbase = jnp.full((LN,), h * 16, jnp.int32)

            @plsc.parallel_loop(0, 16, unroll=8)
            def _col(c):
                colv = hbase + c
                for eb in range(NEB):
                    hvv = plsc.load_gather(hrows.at[slot], [eis[eb], colv])
                    plsc.store_scatter(hrows.at[slot], [eis[eb], colv],
                                       hvv * exh[eb])

            return 0

        lax.fori_loop(0, H, _ph, 0)

    for v, (hv, abv, ev) in enumerate(
            ((h0, ab0, e0), (h1, ab1, e1), (h2, ab2, e2))):
        def _zb(r, _):
            for cc in range(8):
                hrows[0, r, pl.ds(cc * 16, 16)] = zero16
            exden[r, :] = zero16
            return 0

        lax.fori_loop(0, CHUNK, _zb, 0)
        for k in range(RPT // CHUNK):
            pltpu.sync_copy(hrows.at[0],
                            acc_num.at[pl.ds(rows0 + k * CHUNK, CHUNK), :])
            pltpu.sync_copy(exden,
                            acc_den.at[pl.ds(rows0 + k * CHUNK, CHUNK), :])
        plsc.subcore_barrier()

        pltpu.async_copy(ev.at[w, 0], idxg.at[0], si[0])
        pltpu.make_async_copy(ev.at[w, 0], idxg.at[0], si[0]).wait()
        issue_gathers(abv, hv, 0, idxg.at[0, 0, 0], idxg.at[0, 0, 1])

        def _grp2(i, _):
          for gp in range(2):
            g = 2 * i + gp
            s_cur = gp
            s_nxt = 1 - gp
            for b in range(GRP):
                sl = b % 2
                so = 1 - sl
                if b == 0:
                    @pl.when(g > 0)
                    def _():
                        wait_scatters(so, idxg.at[s_nxt, GRP - 1, 1])
                else:
                    wait_scatters(so, idxg.at[s_cur, b - 1, 1])
                if b == 1:
                    @pl.when(g < NGRP - 1)
                    def _():
                        pltpu.async_copy(ev.at[w, g + 1], idxg.at[s_nxt],
                                         si[s_nxt])
                if b == GRP - 1:
                    @pl.when(g < NGRP - 1)
                    def _():
                        pltpu.make_async_copy(ev.at[w, g + 1], idxg.at[s_nxt],
                                              si[s_nxt]).wait()
                        issue_gathers(abv, hv, so, idxg.at[s_nxt, 0, 0],
                                      idxg.at[s_nxt, 0, 1])
                else:
                    issue_gathers(abv, hv, so, idxg.at[s_cur, b + 1, 0],
                                  idxg.at[s_cur, b + 1, 1])
                wait_gathers(abv, hv, sl, idxg.at[s_cur, b, 0],
                             idxg.at[s_cur, b, 1])
                compute_chunk(sl)
                issue_scatters(sl, idxg.at[s_cur, b, 1])
          return 0

        lax.fori_loop(0, NGRP // 2, _grp2, 0)
        wait_scatters((GRP - 1) % 2,
                      idxg.at[(NGRP - 1) % 2, GRP - 1, 1])
        plsc.subcore_barrier()

        for k in range(RPT // CHUNK):
            pltpu.sync_copy(
                acc_num.at[pl.ds(rows0 + k * CHUNK, CHUNK), :],
                num_out.at[v, cid, pl.ds(rows0 + k * CHUNK, CHUNK), :])
        pltpu.sync_copy(acc_den.at[pl.ds(rows0, RPT), :],
                        den_out.at[v, cid, pl.ds(rows0, RPT), :])
        plsc.subcore_barrier()


def _sc_edge(hs, abs_pad, edges):
    mesh = plsc.VectorSubcoreMesh(core_axis_name="c", subcore_axis_name="s")
    f32 = jnp.float32
    kern = pl.kernel(
        _sc_body,
        out_type=(jax.ShapeDtypeStruct((3, NCORE, NP, HC), f32),
                  jax.ShapeDtypeStruct((3, NCORE, NP, 16), f32)),
        mesh=mesh,
        compiler_params=pltpu.CompilerParams(
            needs_layout_passes=False, use_tc_tiling_on_sc=False),
        scratch_types=[
            pltpu.VMEM_SHARED((NP, HC), f32),
            pltpu.VMEM_SHARED((NP, 16), f32),
            pltpu.VMEM((2, GRP, 2, CHUNK), jnp.int32),
            pltpu.VMEM((2, CHUNK, 16), f32),
            pltpu.VMEM((2, CHUNK, 16), f32),
            pltpu.VMEM((2, CHUNK, HC), f32),
            pltpu.VMEM((CHUNK, 16), f32),
            pltpu.SemaphoreType.DMA,
            pltpu.SemaphoreType.DMA,
            pltpu.SemaphoreType.DMA,
            pltpu.SemaphoreType.DMA,
            pltpu.SemaphoreType.DMA,
            pltpu.SemaphoreType.DMA,
        ],
    )
    return kern(*hs, *abs_pad, *edges)


_PB = 1000


def _post_body(num, den, h0, h1, h2, ab0, ab1, ab2, batch3, rep,
               gfeat, b0, b1, b2, lin1w, lin1b, lin2w, lin2b,
               out, accp, acccnt, xcat):
    i = pl.program_id(0)
    nsteps = pl.num_programs(0)

    @pl.when(i == 0)
    def _init():
        accp[...] = jnp.zeros((B, 3 * HC), jnp.float32)
        acccnt[...] = jnp.zeros((B, HC), jnp.float32)
        xcat[...] = jnp.zeros((B, 512), jnp.float32)

    bb = batch3[0]
    iota_b = lax.broadcasted_iota(jnp.int32, (B, _PB), 0)
    oh = (iota_b == bb).astype(jnp.float32)
    acccnt[...] += jnp.sum(oh, axis=1, keepdims=True)

    repm = rep[...]
    for v, (hr, abr) in enumerate(((h0, ab0), (h1, ab1), (h2, ab2))):
        nb = num[v, 0] + num[v, 1]
        db8 = den[v, 0, :, :H] + den[v, 1, :, :H]
        abb = abr[...]
        a8 = abb[:, :H] + abb[:, H:2 * H]
        a128 = jnp.dot(a8, repm, preferred_element_type=jnp.float32)
        el128 = jnp.exp(jnp.where(a128 > 0, a128, a128 * jnp.float32(0.2)))
        d128 = jnp.dot(db8, repm, preferred_element_type=jnp.float32) + el128
        nfull = nb + hr[...] * el128
        outv = nfull / (d128 + jnp.float32(1e-16))
        accp[:, v * HC:(v + 1) * HC] += jnp.dot(
            oh, outv, preferred_element_type=jnp.float32)

    @pl.when(i == nsteps - 1)
    def _fin():
        cnt = acccnt[...]
        denom = jnp.maximum(cnt, 1.0)
        nonempty = (cnt > 0).astype(jnp.float32)
        for v, bv in enumerate((b0, b1, b2)):
            xcat[:, v * HC:(v + 1) * HC] = (
                accp[:, v * HC:(v + 1) * HC] / denom + bv[...] * nonempty)
        xcat[:, 3 * HC:3 * HC + G] = gfeat[...]
        y1 = jnp.dot(xcat[...], lin1w[...], preferred_element_type=jnp.float32)
        y1 = jnp.maximum(y1 + lin1b[...], 0.0)
        y = jnp.dot(y1, lin2w[...], preferred_element_type=jnp.float32)
        out[...] = y + lin2b[...]


def _tc_post(num, den, hs, abs_, batch3, rep, gfeat, biases,
             lin1w_pad, lin1b, lin2w, lin2b):
    grid = (N // _PB,)
    f32 = jnp.float32
    const = lambda shape: pl.BlockSpec(shape, lambda i: tuple(0 for _ in shape))
    in_specs = [
        pl.BlockSpec((3, NCORE, _PB, HC), lambda i: (0, 0, i, 0)),
        pl.BlockSpec((3, NCORE, _PB, 16), lambda i: (0, 0, i, 0)),
    ]
    in_specs += [pl.BlockSpec((_PB, HC), lambda i: (i, 0))] * 3
    in_specs += [pl.BlockSpec((_PB, 16), lambda i: (i, 0))] * 3
    in_specs += [pl.BlockSpec((1, 1, _PB), lambda i: (i, 0, 0))]
    in_specs += [const((H, HC)), const((B, G)),
                 const((1, HC)), const((1, HC)), const((1, HC)),
                 const((512, 16)), const((1, 16)), const((16, NCLS)),
                 const((1, NCLS))]
    out = pl.pallas_call(
        _post_body,
        grid=grid,
        in_specs=in_specs,
        out_specs=pl.BlockSpec((B, NCLS), lambda i: (0, 0)),
        out_shape=jax.ShapeDtypeStruct((B, NCLS), f32),
        scratch_shapes=[
            pltpu.VMEM((B, 3 * HC), f32),
            pltpu.VMEM((B, HC), f32),
            pltpu.VMEM((B, 512), f32),
        ],
    )(num, den, *hs, *abs_, batch3, rep, gfeat, *biases,
      lin1w_pad, lin1b, lin2w, lin2b)
    return out


def kernel(x_anterior, x_posterior, x_lateral, global_features,
           edge_index_anterior, edge_index_posterior, edge_index_lateral,
           batch,
           W_ant, att_src_ant, att_dst_ant, bias_ant,
           W_post, att_src_post, att_dst_post, bias_post,
           W_lat, att_src_lat, att_dst_lat, bias_lat,
           lin1_w, lin1_b, lin2_w, lin2_b):
    f32 = jnp.float32

    def make_ab_mat(att_src, att_dst):
        eye = jnp.eye(H, dtype=f32)
        sel = jnp.repeat(eye, C, axis=0)
        m_src = sel * att_src.reshape(HC, 1)
        m_dst = sel * att_dst.reshape(HC, 1)
        return jnp.concatenate([m_src, m_dst], axis=1)

    ABs = [make_ab_mat(att_src_ant, att_dst_ant),
           make_ab_mat(att_src_post, att_dst_post),
           make_ab_mat(att_src_lat, att_dst_lat)]

    xs = [x_anterior, x_posterior, x_lateral]
    Ws = [W_ant, W_post, W_lat]

    hs, abs_ = _tc_pre(xs, Ws, ABs)

    abs_pad = [jnp.pad(a, ((0, NP - N), (0, 0))) for a in abs_]

    def prep_edges(ei):
        src = jnp.concatenate(
            [ei[0].astype(jnp.int32), jnp.zeros((EPAD - E,), jnp.int32)])
        dst = jnp.concatenate(
            [ei[1].astype(jnp.int32),
             jnp.full((EPAD - E,), N, jnp.int32)])
        src4 = src.reshape(NW, NGRP, GRP, 1, CHUNK)
        dst4 = dst.reshape(NW, NGRP, GRP, 1, CHUNK)
        return jnp.concatenate([src4, dst4], axis=3)

    edges = [prep_edges(edge_index_anterior),
             prep_edges(edge_index_posterior),
             prep_edges(edge_index_lateral)]

    num, den = _sc_edge(hs, abs_pad, edges)

    repm = jnp.repeat(jnp.eye(H, dtype=f32), C, axis=0).T

    batch3 = batch.astype(jnp.int32).reshape(N // _PB, 1, _PB)
    lin1w_pad = jnp.pad(lin1_w, ((0, 512 - lin1_w.shape[0]), (0, 0)))
    biases = [bias_ant.reshape(1, HC), bias_post.reshape(1, HC),
              bias_lat.reshape(1, HC)]

    out = _tc_post(num, den, hs, abs_, batch3, repm,
                   global_features, biases, lin1w_pad,
                   lin1_b.reshape(1, 16), lin2_w, lin2_b.reshape(1, NCLS))
    return out


# speedup vs baseline: 1.2931x; 1.0836x over previous
import functools

import jax
import jax.numpy as jnp
from jax import lax
from jax.experimental import pallas as pl
from jax.experimental.pallas import tpu as pltpu
from jax.experimental.pallas import tpu_sc as plsc

N = 10000
E = 320000
F_IN = 128
H = 8
C = 16
HC = H * C
B = 64
G = 16
NCLS = 10

NCORE = 2
NSUB = 16
LN = 16
NW = NCORE * NSUB

CHUNK = 64
GRP = 4
NGRP = -(-E // (NW * CHUNK * GRP))
NCHUNK = NGRP * GRP
EPW = CHUNK * NCHUNK
EPAD = EPW * NW
NP = 10240
RPT = NP // NSUB
NEB = CHUNK // LN


_BN = 400


def _pre_body(x0, x1, x2, w0, w1, w2, m0, m1, m2,
              h0, h1, h2, ab0, ab1, ab2):
    for x, w, m, h, ab in ((x0, w0, m0, h0, ab0),
                           (x1, w1, m1, h1, ab1),
                           (x2, w2, m2, h2, ab2)):
        hv = jnp.dot(x[...], w[...], preferred_element_type=jnp.float32)
        h[...] = hv
        ab[...] = jnp.dot(hv, m[...], preferred_element_type=jnp.float32)


def _tc_pre(xs, Ws, ABs):
    grid = (N // _BN,)
    xspec = pl.BlockSpec((_BN, F_IN), lambda i: (i, 0))
    wspec = pl.BlockSpec((F_IN, HC), lambda i: (0, 0))
    mspec = pl.BlockSpec((HC, 16), lambda i: (0, 0))
    hspec = pl.BlockSpec((_BN, HC), lambda i: (i, 0))
    abspec = pl.BlockSpec((_BN, 16), lambda i: (i, 0))
    out_shape = ([jax.ShapeDtypeStruct((N, HC), jnp.float32) for _ in range(3)]
                 + [jax.ShapeDtypeStruct((N, 16), jnp.float32) for _ in range(3)])
    res = pl.pallas_call(
        _pre_body,
        grid=grid,
        in_specs=[xspec] * 3 + [wspec] * 3 + [mspec] * 3,
        out_specs=[hspec] * 3 + [abspec] * 3,
        out_shape=out_shape,
    )(*xs, *Ws, *ABs)
    return res[:3], res[3:]


def _sc_body(h0, h1, h2, ab0, ab1, ab2, e0, e1, e2,
             num_out, den_out,
             acc_num, acc_den, idxg, absrc, abdst, hrows,
             exden, sg0, sg1, ss0, ss1, si0, si1):
    cid = lax.axis_index("c")
    sid = lax.axis_index("s")
    w = cid * NSUB + sid
    iota = lax.broadcasted_iota(jnp.int32, (LN,), 0)
    zero16 = jnp.zeros((LN,), jnp.float32)
    rows0 = sid * RPT
    sg = (sg0, sg1)
    ss = (ss0, ss1)
    si = (si0, si1)

    def issue_gathers(abv, hv, slot, src_row, dst_row):
        pltpu.async_copy(abv.at[src_row], absrc.at[slot], sg[slot])
        pltpu.async_copy(abv.at[dst_row], abdst.at[slot], sg[slot])
        pltpu.async_copy(hv.at[src_row], hrows.at[slot], sg[slot])

    def wait_gathers(abv, hv, slot, src_row, dst_row):
        pltpu.make_async_copy(abv.at[src_row], absrc.at[slot],
                              sg[slot]).wait()
        pltpu.make_async_copy(abv.at[dst_row], abdst.at[slot],
                              sg[slot]).wait()
        pltpu.make_async_copy(hv.at[src_row], hrows.at[slot], sg[slot]).wait()

    def issue_scatters(slot, dst_row):
        pltpu.async_copy(hrows.at[slot], acc_num.at[dst_row], ss[slot],
                         add=True)
        pltpu.async_copy(exden, acc_den.at[dst_row], ss[slot], add=True)

    def wait_scatters(slot, dst_row):
        pltpu.make_async_copy(hrows.at[slot], acc_num.at[dst_row],
                              ss[slot]).wait()
        pltpu.make_async_copy(exden, acc_den.at[dst_row], ss[slot]).wait()

    eis = [iota + (eb * LN) for eb in range(NEB)]

    def compute_chunk(slot):
        def _ph(h, _):
            hcol = jnp.full((LN,), h, jnp.int32)
            hcol8 = hcol + 8
            exh = []
            for eb in range(NEB):
                va = plsc.load_gather(absrc.at[slot], [eis[eb], hcol])
                vb = plsc.load_gather(abdst.at[slot], [eis[eb], hcol8])
                al = va + vb
                al = jnp.where(al > 0, al, al * jnp.float32(0.2))
                ex = jnp.exp(al)
                plsc.store_scatter(exden, [eis[eb], hcol], ex)
                exh.append(ex)
            hbase = jnp.full((LN,), h * 16, jnp.int32)

            @plsc.parallel_loop(0, 16, unroll=8)
            def _col(c):
                colv = hbase + c
                for eb in range(NEB):
                    hvv = plsc.load_gather(hrows.at[slot], [eis[eb], colv])
                    plsc.store_scatter(hrows.at[slot], [eis[eb], colv],
                                       hvv * exh[eb])

            return 0

        lax.fori_loop(0, H, _ph, 0)

    for v, (hv, abv, ev) in enumerate(
            ((h0, ab0, e0), (h1, ab1, e1), (h2, ab2, e2))):
        def _zb(r, _):
            for cc in range(8):
                hrows[0, r, pl.ds(cc * 16, 16)] = zero16
            exden[r, :] = zero16
            return 0

        lax.fori_loop(0, CHUNK, _zb, 0)
        for k in range(RPT // CHUNK):
            pltpu.sync_copy(hrows.at[0],
                            acc_num.at[pl.ds(rows0 + k * CHUNK, CHUNK), :])
            pltpu.sync_copy(exden,
                            acc_den.at[pl.ds(rows0 + k * CHUNK, CHUNK), :])
        plsc.subcore_barrier()

        pltpu.async_copy(ev.at[w, 0], idxg.at[0], si[0])
        pltpu.make_async_copy(ev.at[w, 0], idxg.at[0], si[0]).wait()
        issue_gathers(abv, hv, 0, idxg.at[0, 0, 0], idxg.at[0, 0, 1])

        def _grp2(i, _):
          for gp in range(2):
            g = 2 * i + gp
            s_cur = gp
            s_nxt = 1 - gp
            for b in range(GRP):
                sl = b % 2
                so = 1 - sl
                if b == 0:
                    @pl.when(g > 0)
                    def _():
                        wait_scatters(so, idxg.at[s_nxt, GRP - 1, 1])
                else:
                    wait_scatters(so, idxg.at[s_cur, b - 1, 1])
                if b == 1:
                    @pl.when(g < NGRP - 1)
                    def _():
                        pltpu.async_copy(ev.at[w, g + 1], idxg.at[s_nxt],
                                         si[s_nxt])
                if b == GRP - 1:
                    @pl.when(g < NGRP - 1)
                    def _():
                        pltpu.make_async_copy(ev.at[w, g + 1], idxg.at[s_nxt],
                                              si[s_nxt]).wait()
                        issue_gathers(abv, hv, so, idxg.at[s_nxt, 0, 0],
                                      idxg.at[s_nxt, 0, 1])
                else:
                    issue_gathers(abv, hv, so, idxg.at[s_cur, b + 1, 0],
                                  idxg.at[s_cur, b + 1, 1])
                wait_gathers(abv, hv, sl, idxg.at[s_cur, b, 0],
                             idxg.at[s_cur, b, 1])
                compute_chunk(sl)
                issue_scatters(sl, idxg.at[s_cur, b, 1])
          return 0

        lax.fori_loop(0, NGRP // 2, _grp2, 0)
        wait_scatters((GRP - 1) % 2,
                      idxg.at[(NGRP - 1) % 2, GRP - 1, 1])
        plsc.subcore_barrier()

        for k in range(RPT // CHUNK):
            pltpu.sync_copy(
                acc_num.at[pl.ds(rows0 + k * CHUNK, CHUNK), :],
                num_out.at[v, cid, pl.ds(rows0 + k * CHUNK, CHUNK), :])
        pltpu.sync_copy(acc_den.at[pl.ds(rows0, RPT), :],
                        den_out.at[v, cid, pl.ds(rows0, RPT), :])
        plsc.subcore_barrier()


def _sc_edge(hs, abs_pad, edges):
    mesh = plsc.VectorSubcoreMesh(core_axis_name="c", subcore_axis_name="s")
    f32 = jnp.float32
    kern = pl.kernel(
        _sc_body,
        out_type=(jax.ShapeDtypeStruct((3, NCORE, NP, HC), f32),
                  jax.ShapeDtypeStruct((3, NCORE, NP, 16), f32)),
        mesh=mesh,
        compiler_params=pltpu.CompilerParams(
            needs_layout_passes=False, use_tc_tiling_on_sc=False),
        scratch_types=[
            pltpu.VMEM_SHARED((NP, HC), f32),
            pltpu.VMEM_SHARED((NP, 16), f32),
            pltpu.VMEM((2, GRP, 2, CHUNK), jnp.int32),
            pltpu.VMEM((2, CHUNK, 16), f32),
            pltpu.VMEM((2, CHUNK, 16), f32),
            pltpu.VMEM((2, CHUNK, HC), f32),
            pltpu.VMEM((CHUNK, 16), f32),
            pltpu.SemaphoreType.DMA,
            pltpu.SemaphoreType.DMA,
            pltpu.SemaphoreType.DMA,
            pltpu.SemaphoreType.DMA,
            pltpu.SemaphoreType.DMA,
            pltpu.SemaphoreType.DMA,
        ],
    )
    return kern(*hs, *abs_pad, *edges)


_PB = 1000


def _post_body(num, den, h0, h1, h2, ab0, ab1, ab2, batch3, rep,
               gfeat, b0, b1, b2, lin1w, lin1b, lin2w, lin2b,
               out, accp, acccnt, xcat):
    i = pl.program_id(0)
    nsteps = pl.num_programs(0)

    @pl.when(i == 0)
    def _init():
        accp[...] = jnp.zeros((B, 3 * HC), jnp.float32)
        acccnt[...] = jnp.zeros((B, HC), jnp.float32)
        xcat[...] = jnp.zeros((B, 512), jnp.float32)

    bb = batch3[0]
    iota_b = lax.broadcasted_iota(jnp.int32, (B, _PB), 0)
    oh = (iota_b == bb).astype(jnp.float32)
    acccnt[...] += jnp.sum(oh, axis=1, keepdims=True)

    repm = rep[...]
    for v, (hr, abr) in enumerate(((h0, ab0), (h1, ab1), (h2, ab2))):
        nb = num[v, 0] + num[v, 1]
        db8 = den[v, 0, :, :H] + den[v, 1, :, :H]
        abb = abr[...]
        a8 = abb[:, :H] + abb[:, H:2 * H]
        a128 = jnp.dot(a8, repm, preferred_element_type=jnp.float32)
        el128 = jnp.exp(jnp.where(a128 > 0, a128, a128 * jnp.float32(0.2)))
        d128 = jnp.dot(db8, repm, preferred_element_type=jnp.float32) + el128
        nfull = nb + hr[...] * el128
        outv = nfull / (d128 + jnp.float32(1e-16))
        accp[:, v * HC:(v + 1) * HC] += jnp.dot(
            oh, outv, preferred_element_type=jnp.float32)

    @pl.when(i == nsteps - 1)
    def _fin():
        cnt = acccnt[...]
        denom = jnp.maximum(cnt, 1.0)
        nonempty = (cnt > 0).astype(jnp.float32)
        for v, bv in enumerate((b0, b1, b2)):
            xcat[:, v * HC:(v + 1) * HC] = (
                accp[:, v * HC:(v + 1) * HC] / denom + bv[...] * nonempty)
        xcat[:, 3 * HC:3 * HC + G] = gfeat[...]
        y1 = jnp.dot(xcat[...], lin1w[...], preferred_element_type=jnp.float32)
        y1 = jnp.maximum(y1 + lin1b[...], 0.0)
        y = jnp.dot(y1, lin2w[...], preferred_element_type=jnp.float32)
        out[...] = y + lin2b[...]


def _tc_post(num, den, hs, abs_, batch3, rep, gfeat, biases,
             lin1w_pad, lin1b, lin2w, lin2b):
    grid = (N // _PB,)
    f32 = jnp.float32
    const = lambda shape: pl.BlockSpec(shape, lambda i: tuple(0 for _ in shape))
    in_specs = [
        pl.BlockSpec((3, NCORE, _PB, HC), lambda i: (0, 0, i, 0)),
        pl.BlockSpec((3, NCORE, _PB, 16), lambda i: (0, 0, i, 0)),
    ]
    in_specs += [pl.BlockSpec((_PB, HC), lambda i: (i, 0))] * 3
    in_specs += [pl.BlockSpec((_PB, 16), lambda i: (i, 0))] * 3
    in_specs += [pl.BlockSpec((1, 1, _PB), lambda i: (i, 0, 0))]
    in_specs += [const((H, HC)), const((B, G)),
                 const((1, HC)), const((1, HC)), const((1, HC)),
                 const((512, 16)), const((1, 16)), const((16, NCLS)),
                 const((1, NCLS))]
    out = pl.pallas_call(
        _post_body,
        grid=grid,
        in_specs=in_specs,
        out_specs=pl.BlockSpec((B, NCLS), lambda i: (0, 0)),
        out_shape=jax.ShapeDtypeStruct((B, NCLS), f32),
        scratch_shapes=[
            pltpu.VMEM((B, 3 * HC), f32),
            pltpu.VMEM((B, HC), f32),
            pltpu.VMEM((B, 512), f32),
        ],
    )(num, den, *hs, *abs_, batch3, rep, gfeat, *biases,
      lin1w_pad, lin1b, lin2w, lin2b)
    return out


def kernel(x_anterior, x_posterior, x_lateral, global_features,
           edge_index_anterior, edge_index_posterior, edge_index_lateral,
           batch,
           W_ant, att_src_ant, att_dst_ant, bias_ant,
           W_post, att_src_post, att_dst_post, bias_post,
           W_lat, att_src_lat, att_dst_lat, bias_lat,
           lin1_w, lin1_b, lin2_w, lin2_b):
    f32 = jnp.float32

    def make_ab_mat(att_src, att_dst):
        eye = jnp.eye(H, dtype=f32)
        sel = jnp.repeat(eye, C, axis=0)
        m_src = sel * att_src.reshape(HC, 1)
        m_dst = sel * att_dst.reshape(HC, 1)
        return jnp.concatenate([m_src, m_dst], axis=1)

    ABs = [make_ab_mat(att_src_ant, att_dst_ant),
           make_ab_mat(att_src_post, att_dst_post),
           make_ab_mat(att_src_lat, att_dst_lat)]

    xs = [x_anterior, x_posterior, x_lateral]
    Ws = [W_ant, W_post, W_lat]

    hs, abs_ = _tc_pre(xs, Ws, ABs)

    abs_pad = [jnp.pad(a, ((0, NP - N), (0, 0))) for a in abs_]

    def prep_edges(ei):
        src = jnp.concatenate(
            [ei[0].astype(jnp.int32), jnp.zeros((EPAD - E,), jnp.int32)])
        dst = jnp.concatenate(
            [ei[1].astype(jnp.int32),
             jnp.full((EPAD - E,), N, jnp.int32)])
        src4 = src.reshape(NW, NGRP, GRP, 1, CHUNK)
        dst4 = dst.reshape(NW, NGRP, GRP, 1, CHUNK)
        return jnp.concatenate([src4, dst4], axis=3)

    edges = [prep_edges(edge_index_anterior),
             prep_edges(edge_index_posterior),
             prep_edges(edge_index_lateral)]

    num, den = _sc_edge(hs, abs_pad, edges)

    repm = jnp.repeat(jnp.eye(H, dtype=f32), C, axis=0).T

    batch3 = batch.astype(jnp.int32).reshape(N // _PB, 1, _PB)
    lin1w_pad = jnp.pad(lin1_w, ((0, 512 - lin1_w.shape[0]), (0, 0)))
    biases = [bias_ant.reshape(1, HC), bias_post.reshape(1, HC),
              bias_lat.reshape(1, HC)]

    out = _tc_post(num, den, hs, abs_, batch3, repm,
                   global_features, biases, lin1w_pad,
                   lin1_b.reshape(1, 16), lin2_w, lin2_b.reshape(1, NCLS))
    return out
